# baseline (speedup 1.0000x reference)
import functools

import jax
import jax.numpy as jnp
from jax import lax
from jax.experimental import pallas as pl
from jax.experimental.pallas import tpu as pltpu
from jax.experimental.pallas import tpu_sc as plsc

_B = 16384
_NF = 8
_DM = 32


@functools.cache
def _make_sc_gather():
    info = plsc.get_sparse_core_info()
    nc, ns = info.num_cores, info.num_subcores
    nw = nc * ns
    bpw = _B // nw
    mesh = plsc.VectorSubcoreMesh(core_axis_name="c", subcore_axis_name="s")

    @functools.partial(
        pl.kernel,
        mesh=mesh,
        compiler_params=pltpu.CompilerParams(use_tc_tiling_on_sc=False),
        out_type=[
            jax.ShapeDtypeStruct((_B, _NF), jnp.float32),
            jax.ShapeDtypeStruct((_B, _NF), jnp.float32),
            jax.ShapeDtypeStruct((_B, _DM), jnp.float32),
            jax.ShapeDtypeStruct((_B, _DM), jnp.float32),
        ],
        scratch_types=[
            pltpu.VMEM((bpw,), jnp.int32),
            pltpu.VMEM((bpw,), jnp.int32),
            pltpu.VMEM((bpw, _NF), jnp.float32),
            pltpu.VMEM((bpw, _NF), jnp.float32),
            pltpu.VMEM((bpw, _DM), jnp.float32),
            pltpu.VMEM((bpw, _DM), jnp.float32),
            pltpu.SemaphoreType.DMA,
            pltpu.SemaphoreType.DMA,
            pltpu.SemaphoreType.DMA,
            pltpu.SemaphoreType.DMA,
        ],
    )
    def gather(users, items, Ug, Ig, Um, Im,
               ug_o, ig_o, eu_o, ei_o,
               uidx, iidx, ugv, igv, euv, eiv, s0, s1, s2, s3):
        wid = lax.axis_index("s") * nc + lax.axis_index("c")
        base = wid * bpw
        pltpu.sync_copy(users.at[pl.ds(base, bpw)], uidx)
        pltpu.sync_copy(items.at[pl.ds(base, bpw)], iidx)
        c0 = pltpu.async_copy(Ug.at[uidx], ugv, s0)
        c1 = pltpu.async_copy(Ig.at[iidx], igv, s1)
        c2 = pltpu.async_copy(Um.at[uidx], euv, s2)
        c3 = pltpu.async_copy(Im.at[iidx], eiv, s3)
        c0.wait()
        c1.wait()
        c2.wait()
        c3.wait()
        pltpu.sync_copy(ugv, ug_o.at[pl.ds(base, bpw)])
        pltpu.sync_copy(igv, ig_o.at[pl.ds(base, bpw)])
        pltpu.sync_copy(euv, eu_o.at[pl.ds(base, bpw)])
        pltpu.sync_copy(eiv, ei_o.at[pl.ds(base, bpw)])

    return gather


_BLK = 2048


def _mlp_body(ug_r, ig_r, eu_r, ei_r, w1_r, b1_r, w2_r, b2_r, w3_r, b3_r,
              wl_r, bl_r, o_r):
    dn = (((1,), (1,)), ((), ()))
    f32 = jnp.float32
    g = ug_r[...] * ig_r[...]
    w1 = w1_r[...]
    h = lax.dot_general(eu_r[...], w1[:, :_DM], dn, preferred_element_type=f32)
    h = h + lax.dot_general(ei_r[...], w1[:, _DM:], dn, preferred_element_type=f32)
    h = jnp.maximum(h + b1_r[...], 0.0)
    h = lax.dot_general(h, w2_r[...], dn, preferred_element_type=f32)
    h = jnp.maximum(h + b2_r[...], 0.0)
    h = lax.dot_general(h, w3_r[...], dn, preferred_element_type=f32)
    h = jnp.maximum(h + b3_r[...], 0.0)
    wl = wl_r[...]
    out = lax.dot_general(g, wl[:, :_NF], dn, preferred_element_type=f32)
    out = out + lax.dot_general(h, wl[:, _NF:], dn, preferred_element_type=f32)
    o_r[...] = out + bl_r[...]


def _mlp(ug, ig, eu, ei, W1, b1, W2, b2, W3, b3, Wl, bl):
    def full(shape):
        nd = len(shape)
        return pl.BlockSpec(shape, lambda i: (0,) * nd)

    grid = _B // _BLK
    return pl.pallas_call(
        _mlp_body,
        grid=(grid,),
        in_specs=[
            pl.BlockSpec((_BLK, _NF), lambda i: (i, 0)),
            pl.BlockSpec((_BLK, _NF), lambda i: (i, 0)),
            pl.BlockSpec((_BLK, _DM), lambda i: (i, 0)),
            pl.BlockSpec((_BLK, _DM), lambda i: (i, 0)),
            full(W1.shape), full((1, 32)), full(W2.shape), full((1, 16)),
            full(W3.shape), full((1, 8)), full(Wl.shape), full((1, 1)),
        ],
        out_specs=pl.BlockSpec((_BLK, 1), lambda i: (i, 0)),
        out_shape=jax.ShapeDtypeStruct((_B, 1), jnp.float32),
    )(ug, ig, eu, ei, W1, b1.reshape(1, -1), W2, b2.reshape(1, -1),
      W3, b3.reshape(1, -1), Wl, bl.reshape(1, -1))


def kernel(users, items, Ug, Ig, Um, Im, W1, b1, W2, b2, W3, b3, Wl, bl):
    ug, ig, eu, ei = _make_sc_gather()(users, items, Ug, Ig, Um, Im)
    out = _mlp(ug, ig, eu, ei, W1, b1, W2, b2, W3, b3, Wl, bl)
    return out.reshape(-1)
